# baseline (speedup 1.0000x reference)
import functools

import jax
import jax.numpy as jnp
from jax import lax
from jax.experimental import pallas as pl
from jax.experimental.pallas import tpu as pltpu
from jax.experimental.pallas import tpu_sc as plsc

VOCAB = 30522
HIDDEN = 768
BATCH = 4
SEQ = 2048
EPS = 1e-12
L = 16
HC = HIDDEN // L
CH = 16
NBUF = 4
UNROLL = 12


def _hsum(v):
    s = [v[i] for i in range(L)]
    while len(s) > 1:
        s = [s[2 * i] + s[2 * i + 1] for i in range(len(s) // 2)]
    return s[0]


def _rsqrt(x):
    i = lax.bitcast_convert_type(x, jnp.int32)
    i = jnp.int32(0x5F3759DF) - lax.shift_right_logical(i, jnp.int32(1))
    y = lax.bitcast_convert_type(i, jnp.float32)
    for _ in range(3):
        y = y * (1.5 - 0.5 * x * y * y)
    return y


def _body(nc, spt, ids_hbm, ttb_hbm, word_hbm, pos_hbm, type_hbm, gamma_hbm,
          beta_hbm, out_hbm, idx_v, ttb_v, r0, r1, r2, r3, x_v, pos_v,
          type_v, diff_v, gamma_v, beta_v, sg0, sg1, sg2, sg3, so0, so1,
          so2, so3, sstage):
    wid = lax.axis_index("s") * nc + lax.axis_index("c")
    s0 = wid * spt
    qpb = spt // CH
    nchunks = BATCH * qpb

    staging = []
    for b in range(BATCH):
        staging.append(pltpu.async_copy(ids_hbm.at[b, pl.ds(s0, spt)],
                                        idx_v.at[pl.ds(b * spt, spt)],
                                        sstage))
        staging.append(pltpu.async_copy(ttb_hbm.at[b, pl.ds(s0 * L, spt * L)],
                                        ttb_v.at[b], sstage))
    staging.append(pltpu.async_copy(pos_hbm.at[pl.ds(s0, spt)], pos_v,
                                    sstage))
    staging.append(pltpu.async_copy(type_hbm, type_v, sstage))
    staging.append(pltpu.async_copy(gamma_hbm, gamma_v, sstage))
    staging.append(pltpu.async_copy(beta_hbm, beta_v, sstage))
    for h in staging:
        h.wait()

    for c in range(HC):
        o = pl.ds(c * L, L)
        diff_v[o] = type_v[1, o] - type_v[0, o]

    @plsc.parallel_loop(0, spt, unroll=2)
    def ploop(i):
        for c in range(HC):
            o = pl.ds(c * L, L)
            pos_v[i, o] = pos_v[i, o] + type_v[0, o]

    rows = (r0, r1, r2, r3)
    sem_g = (sg0, sg1, sg2, sg3)
    sem_o = (so0, so1, so2, so3)

    def gather(k, slot):
        b, q = k // qpb, k % qpb
        return pltpu.async_copy(
            word_hbm.at[idx_v.at[pl.ds(b * spt + q * CH, CH)]], rows[slot],
            sem_g[slot])

    def compute(k, slot):
        rv = rows[slot]
        b, q = k // qpb, k % qpb
        poff = q * CH
        zero = jnp.zeros((L,), jnp.float32)
        nacc = 4

        @plsc.parallel_loop(0, CH, unroll=1)
        def tok(j):
            ttf = ttb_v[b, pl.ds((poff + j) * L, L)]

            @plsc.parallel_loop(0, HC, step=8, carry=(zero,) * (2 * nacc))
            def p1(c0, carry):
                acc = list(carry)
                b0 = c0 * L
                for a in range(8):
                    o = pl.ds(b0 + a * L, L)
                    x = rv[j, o] + pos_v[poff + j, o] + ttf * diff_v[o]
                    x_v[j, o] = x
                    i = a % nacc
                    acc[i] = acc[i] + x
                    acc[nacc + i] = acc[nacc + i] + x * x
                return tuple(acc)

            acc = list(p1)
            while len(acc) > 2:
                acc = ([acc[2 * i] + acc[2 * i + 1]
                        for i in range(len(acc) // 4)]
                       + [acc[len(acc) // 2 + 2 * i]
                          + acc[len(acc) // 2 + 2 * i + 1]
                          for i in range(len(acc) // 4)])
            vs, vq = acc
            mean = _hsum(vs) * (1.0 / HIDDEN)
            var = _hsum(vq) * (1.0 / HIDDEN) - mean * mean
            rstd = _rsqrt(var + EPS)
            gm = rstd * mean

            @plsc.parallel_loop(0, HC, step=8)
            def p2(c0):
                b0 = c0 * L
                for a in range(8):
                    o = pl.ds(b0 + a * L, L)
                    x = x_v[j, o]
                    rv[j, o] = ((x * rstd - gm) * gamma_v[o] + beta_v[o])

    def writeback(k, slot):
        b, q = k // qpb, k % qpb
        return pltpu.async_copy(
            rows[slot], out_hbm.at[b, pl.ds(s0 + q * CH, CH)], sem_o[slot])

    gh = [None] * NBUF
    oh = [None] * NBUF
    for k in range(min(NBUF - 1, nchunks)):
        gh[k] = gather(k, k)
    for k in range(nchunks):
        slot = k % NBUF
        gh[slot].wait()
        kn = k + NBUF - 1
        if kn < nchunks:
            sn = kn % NBUF
            if oh[sn] is not None:
                oh[sn].wait()
            gh[sn] = gather(kn, sn)
        compute(k, slot)
        oh[slot] = writeback(k, slot)
    for h in oh:
        if h is not None:
            h.wait()


def kernel(input_ids, token_type_ids, word_emb, pos_emb, type_emb, ln_gamma,
           ln_beta):
    ids = input_ids.astype(jnp.int32)
    ttb = jnp.broadcast_to(token_type_ids.astype(jnp.float32)[..., None],
                           (BATCH, SEQ, L)).reshape(BATCH, SEQ * L)

    try:
        info = plsc.get_sparse_core_info()
        nc, ns = info.num_cores, info.num_subcores
    except Exception:
        nc, ns = 2, 16
    nw = nc * ns
    spt = SEQ // nw

    f = pl.kernel(
        functools.partial(_body, nc, spt),
        out_type=jax.ShapeDtypeStruct((BATCH, SEQ, HIDDEN), jnp.float32),
        mesh=plsc.VectorSubcoreMesh(core_axis_name="c", subcore_axis_name="s"),
        scratch_types=[
            pltpu.VMEM((BATCH * spt,), jnp.int32),
            pltpu.VMEM((BATCH, spt * L), jnp.float32),
            pltpu.VMEM((CH, HIDDEN), jnp.float32),
            pltpu.VMEM((CH, HIDDEN), jnp.float32),
            pltpu.VMEM((CH, HIDDEN), jnp.float32),
            pltpu.VMEM((CH, HIDDEN), jnp.float32),
            pltpu.VMEM((CH, HIDDEN), jnp.float32),
            pltpu.VMEM((spt, HIDDEN), jnp.float32),
            pltpu.VMEM((2, HIDDEN), jnp.float32),
            pltpu.VMEM((HIDDEN,), jnp.float32),
            pltpu.VMEM((HIDDEN,), jnp.float32),
            pltpu.VMEM((HIDDEN,), jnp.float32),
            pltpu.SemaphoreType.DMA,
            pltpu.SemaphoreType.DMA,
            pltpu.SemaphoreType.DMA,
            pltpu.SemaphoreType.DMA,
            pltpu.SemaphoreType.DMA,
            pltpu.SemaphoreType.DMA,
            pltpu.SemaphoreType.DMA,
            pltpu.SemaphoreType.DMA,
            pltpu.SemaphoreType.DMA,
        ],
    )
    return f(ids, ttb, word_emb, pos_emb, type_emb, ln_gamma, ln_beta)


# speedup vs baseline: 1.6536x; 1.6536x over previous
import functools

import jax
import jax.numpy as jnp
from jax import lax
from jax.experimental import pallas as pl
from jax.experimental.pallas import tpu as pltpu
from jax.experimental.pallas import tpu_sc as plsc

VOCAB = 30522
HIDDEN = 768
BATCH = 4
SEQ = 2048
EPS = 1e-12
L = 16
HC = HIDDEN // L
CH = 16
NBUF = 4
UNROLL = 12


def _hsum(v):
    s = [v[i] for i in range(L)]
    while len(s) > 1:
        s = [s[2 * i] + s[2 * i + 1] for i in range(len(s) // 2)]
    return s[0]


def _rsqrt(x):
    i = lax.bitcast_convert_type(x, jnp.int32)
    i = jnp.int32(0x5F3759DF) - lax.shift_right_logical(i, jnp.int32(1))
    y = lax.bitcast_convert_type(i, jnp.float32)
    for _ in range(3):
        y = y * (1.5 - 0.5 * x * y * y)
    return y


def _body(nc, spt, ids_hbm, ttb_hbm, word_hbm, pos_hbm, type_hbm, gamma_hbm,
          beta_hbm, out_hbm, idx_v, ttb_v, r0, r1, r2, r3, x_v, pos_v,
          type_v, diff_v, gamma_v, beta_v, sg0, sg1, sg2, sg3, so0, so1,
          so2, so3, sstage):
    wid = lax.axis_index("s") * nc + lax.axis_index("c")
    s0 = wid * spt
    qpb = spt // CH
    nchunks = BATCH * qpb

    staging = []
    for b in range(BATCH):
        staging.append(pltpu.async_copy(ids_hbm.at[b, pl.ds(s0, spt)],
                                        idx_v.at[pl.ds(b * spt, spt)],
                                        sstage))
        staging.append(pltpu.async_copy(ttb_hbm.at[b, pl.ds(s0 * L, spt * L)],
                                        ttb_v.at[b], sstage))
    staging.append(pltpu.async_copy(pos_hbm.at[pl.ds(s0, spt)], pos_v,
                                    sstage))
    staging.append(pltpu.async_copy(type_hbm, type_v, sstage))
    staging.append(pltpu.async_copy(gamma_hbm, gamma_v, sstage))
    staging.append(pltpu.async_copy(beta_hbm, beta_v, sstage))
    for h in staging:
        h.wait()

    for c in range(HC):
        o = pl.ds(c * L, L)
        diff_v[o] = type_v[1, o] - type_v[0, o]

    @plsc.parallel_loop(0, spt, unroll=2)
    def ploop(i):
        for c in range(HC):
            o = pl.ds(c * L, L)
            pos_v[i, o] = pos_v[i, o] + type_v[0, o]

    rows = (r0, r1, r2, r3)
    sem_g = (sg0, sg1, sg2, sg3)
    sem_o = (so0, so1, so2, so3)

    def gather(k, slot):
        b, q = k // qpb, k % qpb
        return pltpu.async_copy(
            word_hbm.at[idx_v.at[pl.ds(b * spt + q * CH, CH)]], rows[slot],
            sem_g[slot])

    def compute(k, slot):
        rv = rows[slot]
        b, q = k // qpb, k % qpb
        poff = q * CH
        zero = jnp.zeros((L,), jnp.float32)
        nacc = 4

        @plsc.parallel_loop(0, CH, unroll=1)
        def tok(j):
            ttf = ttb_v[b, pl.ds((poff + j) * L, L)]

            @plsc.parallel_loop(0, HC, step=8, carry=(zero,) * (2 * nacc))
            def p1(c0, carry):
                acc = list(carry)
                b0 = c0 * L
                os_ = [pl.ds(b0 + a * L, L) for a in range(8)]
                ws = [rv[j, o] for o in os_]
                ps = [pos_v[poff + j, o] for o in os_]
                ds_ = [diff_v[o] for o in os_]
                xs = [w + p + ttf * d for w, p, d in zip(ws, ps, ds_)]
                for a, (o, x) in enumerate(zip(os_, xs)):
                    x_v[j, o] = x
                    i = a % nacc
                    acc[i] = acc[i] + x
                    acc[nacc + i] = acc[nacc + i] + x * x
                return tuple(acc)

            acc = list(p1)
            while len(acc) > 2:
                acc = ([acc[2 * i] + acc[2 * i + 1]
                        for i in range(len(acc) // 4)]
                       + [acc[len(acc) // 2 + 2 * i]
                          + acc[len(acc) // 2 + 2 * i + 1]
                          for i in range(len(acc) // 4)])
            vs, vq = acc
            mean = _hsum(vs) * (1.0 / HIDDEN)
            var = _hsum(vq) * (1.0 / HIDDEN) - mean * mean
            rstd = _rsqrt(var + EPS)
            gm = rstd * mean

            @plsc.parallel_loop(0, HC, step=8)
            def p2(c0):
                b0 = c0 * L
                os_ = [pl.ds(b0 + a * L, L) for a in range(8)]
                xs = [x_v[j, o] for o in os_]
                gs = [gamma_v[o] for o in os_]
                bs = [beta_v[o] for o in os_]
                ys = [(x * rstd - gm) * g + bb
                      for x, g, bb in zip(xs, gs, bs)]
                for o, y in zip(os_, ys):
                    rv[j, o] = y

    def writeback(k, slot):
        b, q = k // qpb, k % qpb
        return pltpu.async_copy(
            rows[slot], out_hbm.at[b, pl.ds(s0 + q * CH, CH)], sem_o[slot])

    gh = [None] * NBUF
    oh = [None] * NBUF
    for k in range(min(NBUF - 1, nchunks)):
        gh[k] = gather(k, k)
    for k in range(nchunks):
        slot = k % NBUF
        gh[slot].wait()
        kn = k + NBUF - 1
        if kn < nchunks:
            sn = kn % NBUF
            if oh[sn] is not None:
                oh[sn].wait()
            gh[sn] = gather(kn, sn)
        compute(k, slot)
        oh[slot] = writeback(k, slot)
    for h in oh:
        if h is not None:
            h.wait()


def kernel(input_ids, token_type_ids, word_emb, pos_emb, type_emb, ln_gamma,
           ln_beta):
    ids = input_ids.astype(jnp.int32)
    ttb = jnp.broadcast_to(token_type_ids.astype(jnp.float32)[..., None],
                           (BATCH, SEQ, L)).reshape(BATCH, SEQ * L)

    try:
        info = plsc.get_sparse_core_info()
        nc, ns = info.num_cores, info.num_subcores
    except Exception:
        nc, ns = 2, 16
    nw = nc * ns
    spt = SEQ // nw

    f = pl.kernel(
        functools.partial(_body, nc, spt),
        out_type=jax.ShapeDtypeStruct((BATCH, SEQ, HIDDEN), jnp.float32),
        mesh=plsc.VectorSubcoreMesh(core_axis_name="c", subcore_axis_name="s"),
        scratch_types=[
            pltpu.VMEM((BATCH * spt,), jnp.int32),
            pltpu.VMEM((BATCH, spt * L), jnp.float32),
            pltpu.VMEM((CH, HIDDEN), jnp.float32),
            pltpu.VMEM((CH, HIDDEN), jnp.float32),
            pltpu.VMEM((CH, HIDDEN), jnp.float32),
            pltpu.VMEM((CH, HIDDEN), jnp.float32),
            pltpu.VMEM((CH, HIDDEN), jnp.float32),
            pltpu.VMEM((spt, HIDDEN), jnp.float32),
            pltpu.VMEM((2, HIDDEN), jnp.float32),
            pltpu.VMEM((HIDDEN,), jnp.float32),
            pltpu.VMEM((HIDDEN,), jnp.float32),
            pltpu.VMEM((HIDDEN,), jnp.float32),
            pltpu.SemaphoreType.DMA,
            pltpu.SemaphoreType.DMA,
            pltpu.SemaphoreType.DMA,
            pltpu.SemaphoreType.DMA,
            pltpu.SemaphoreType.DMA,
            pltpu.SemaphoreType.DMA,
            pltpu.SemaphoreType.DMA,
            pltpu.SemaphoreType.DMA,
            pltpu.SemaphoreType.DMA,
        ],
    )
    return f(ids, ttb, word_emb, pos_emb, type_emb, ln_gamma, ln_beta)


# speedup vs baseline: 1.6717x; 1.0110x over previous
import functools

import jax
import jax.numpy as jnp
from jax import lax
from jax.experimental import pallas as pl
from jax.experimental.pallas import tpu as pltpu
from jax.experimental.pallas import tpu_sc as plsc

VOCAB = 30522
HIDDEN = 768
BATCH = 4
SEQ = 2048
EPS = 1e-12
L = 16
HC = HIDDEN // L
CH = 16
NBUF = 4
UNROLL = 12


def _hsum(v):
    s = [v[i] for i in range(L)]
    while len(s) > 1:
        s = [s[2 * i] + s[2 * i + 1] for i in range(len(s) // 2)]
    return s[0]


def _rsqrt(x):
    i = lax.bitcast_convert_type(x, jnp.int32)
    i = jnp.int32(0x5F3759DF) - lax.shift_right_logical(i, jnp.int32(1))
    y = lax.bitcast_convert_type(i, jnp.float32)
    for _ in range(3):
        y = y * (1.5 - 0.5 * x * y * y)
    return y


def _body(nc, spt, ids_hbm, ttb_hbm, word_hbm, pos_hbm, type_hbm, gamma_hbm,
          beta_hbm, out_hbm, idx_v, ttb_v, r0, r1, r2, r3, x_v, pos_v,
          type_v, diff_v, gamma_v, beta_v, sg0, sg1, sg2, sg3, so0, so1,
          so2, so3, sstage):
    wid = lax.axis_index("s") * nc + lax.axis_index("c")
    s0 = wid * spt
    qpb = spt // CH
    nchunks = BATCH * qpb

    id_stage = [pltpu.async_copy(ids_hbm.at[b, pl.ds(s0, spt)],
                                 idx_v.at[pl.ds(b * spt, spt)], sstage)
                for b in range(BATCH)]
    staging = [pltpu.async_copy(ttb_hbm.at[b, pl.ds(s0 * L, spt * L)],
                                ttb_v.at[b], sstage) for b in range(BATCH)]
    staging.append(pltpu.async_copy(pos_hbm.at[pl.ds(s0, spt)], pos_v,
                                    sstage))
    staging.append(pltpu.async_copy(type_hbm, type_v, sstage))
    staging.append(pltpu.async_copy(gamma_hbm, gamma_v, sstage))
    staging.append(pltpu.async_copy(beta_hbm, beta_v, sstage))
    for h in id_stage:
        h.wait()

    rows = (r0, r1, r2, r3)
    sem_g = (sg0, sg1, sg2, sg3)
    sem_o = (so0, so1, so2, so3)

    def gather(k, slot):
        b, q = k // qpb, k % qpb
        return pltpu.async_copy(
            word_hbm.at[idx_v.at[pl.ds(b * spt + q * CH, CH)]], rows[slot],
            sem_g[slot])

    gh = [None] * NBUF
    for k in range(min(NBUF - 1, BATCH * qpb)):
        gh[k] = gather(k, k)
    for h in staging:
        h.wait()

    for c in range(HC):
        o = pl.ds(c * L, L)
        diff_v[o] = type_v[1, o] - type_v[0, o]

    @plsc.parallel_loop(0, spt)
    def ploop(i):
        for g in range(0, HC, 8):
            os_ = [pl.ds(c * L, L) for c in range(g, g + 8)]
            ps = [pos_v[i, o] for o in os_]
            ts = [type_v[0, o] for o in os_]
            for o, p, t in zip(os_, ps, ts):
                pos_v[i, o] = p + t

    def compute(k, slot):
        rv = rows[slot]
        b, q = k // qpb, k % qpb
        poff = q * CH
        zero = jnp.zeros((L,), jnp.float32)
        nacc = 4

        @plsc.parallel_loop(0, CH, unroll=1)
        def tok(j):
            ttf = ttb_v[b, pl.ds((poff + j) * L, L)]

            @plsc.parallel_loop(0, HC, step=8, carry=(zero,) * (2 * nacc))
            def p1(c0, carry):
                acc = list(carry)
                b0 = c0 * L
                os_ = [pl.ds(b0 + a * L, L) for a in range(8)]
                ws = [rv[j, o] for o in os_]
                ps = [pos_v[poff + j, o] for o in os_]
                ds_ = [diff_v[o] for o in os_]
                xs = [w + p + ttf * d for w, p, d in zip(ws, ps, ds_)]
                for a, (o, x) in enumerate(zip(os_, xs)):
                    x_v[j, o] = x
                    i = a % nacc
                    acc[i] = acc[i] + x
                    acc[nacc + i] = acc[nacc + i] + x * x
                return tuple(acc)

            acc = list(p1)
            while len(acc) > 2:
                acc = ([acc[2 * i] + acc[2 * i + 1]
                        for i in range(len(acc) // 4)]
                       + [acc[len(acc) // 2 + 2 * i]
                          + acc[len(acc) // 2 + 2 * i + 1]
                          for i in range(len(acc) // 4)])
            vs, vq = acc
            mean = _hsum(vs) * (1.0 / HIDDEN)
            var = _hsum(vq) * (1.0 / HIDDEN) - mean * mean
            rstd = _rsqrt(var + EPS)
            gm = rstd * mean

            @plsc.parallel_loop(0, HC, step=8)
            def p2(c0):
                b0 = c0 * L
                os_ = [pl.ds(b0 + a * L, L) for a in range(8)]
                xs = [x_v[j, o] for o in os_]
                gs = [gamma_v[o] for o in os_]
                bs = [beta_v[o] for o in os_]
                ys = [(x * rstd - gm) * g + bb
                      for x, g, bb in zip(xs, gs, bs)]
                for o, y in zip(os_, ys):
                    rv[j, o] = y

    def writeback(k, slot):
        b, q = k // qpb, k % qpb
        return pltpu.async_copy(
            rows[slot], out_hbm.at[b, pl.ds(s0 + q * CH, CH)], sem_o[slot])

    oh = [None] * NBUF
    for k in range(nchunks):
        slot = k % NBUF
        gh[slot].wait()
        kn = k + NBUF - 1
        if kn < nchunks:
            sn = kn % NBUF
            if oh[sn] is not None:
                oh[sn].wait()
            gh[sn] = gather(kn, sn)
        compute(k, slot)
        oh[slot] = writeback(k, slot)
    for h in oh:
        if h is not None:
            h.wait()


def kernel(input_ids, token_type_ids, word_emb, pos_emb, type_emb, ln_gamma,
           ln_beta):
    ids = input_ids.astype(jnp.int32)
    ttb = jnp.broadcast_to(token_type_ids.astype(jnp.float32)[..., None],
                           (BATCH, SEQ, L)).reshape(BATCH, SEQ * L)

    try:
        info = plsc.get_sparse_core_info()
        nc, ns = info.num_cores, info.num_subcores
    except Exception:
        nc, ns = 2, 16
    nw = nc * ns
    spt = SEQ // nw

    f = pl.kernel(
        functools.partial(_body, nc, spt),
        out_type=jax.ShapeDtypeStruct((BATCH, SEQ, HIDDEN), jnp.float32),
        mesh=plsc.VectorSubcoreMesh(core_axis_name="c", subcore_axis_name="s"),
        scratch_types=[
            pltpu.VMEM((BATCH * spt,), jnp.int32),
            pltpu.VMEM((BATCH, spt * L), jnp.float32),
            pltpu.VMEM((CH, HIDDEN), jnp.float32),
            pltpu.VMEM((CH, HIDDEN), jnp.float32),
            pltpu.VMEM((CH, HIDDEN), jnp.float32),
            pltpu.VMEM((CH, HIDDEN), jnp.float32),
            pltpu.VMEM((CH, HIDDEN), jnp.float32),
            pltpu.VMEM((spt, HIDDEN), jnp.float32),
            pltpu.VMEM((2, HIDDEN), jnp.float32),
            pltpu.VMEM((HIDDEN,), jnp.float32),
            pltpu.VMEM((HIDDEN,), jnp.float32),
            pltpu.VMEM((HIDDEN,), jnp.float32),
            pltpu.SemaphoreType.DMA,
            pltpu.SemaphoreType.DMA,
            pltpu.SemaphoreType.DMA,
            pltpu.SemaphoreType.DMA,
            pltpu.SemaphoreType.DMA,
            pltpu.SemaphoreType.DMA,
            pltpu.SemaphoreType.DMA,
            pltpu.SemaphoreType.DMA,
            pltpu.SemaphoreType.DMA,
        ],
    )
    return f(ids, ttb, word_emb, pos_emb, type_emb, ln_gamma, ln_beta)


# speedup vs baseline: 1.7231x; 1.0308x over previous
import functools

import jax
import jax.numpy as jnp
from jax import lax
from jax.experimental import pallas as pl
from jax.experimental.pallas import tpu as pltpu
from jax.experimental.pallas import tpu_sc as plsc

VOCAB = 30522
HIDDEN = 768
BATCH = 4
SEQ = 2048
EPS = 1e-12
L = 16
HC = HIDDEN // L
CH = 16
NBUF = 4
UNROLL = 12


def _hsum(v):
    lanes = jnp.arange(L, dtype=jnp.int32)
    for sh in (8, 4, 2, 1):
        idx = (lanes + sh) & (L - 1)
        v = v + v.at[idx].get(mode="promise_in_bounds")
    return v[0]


def _rsqrt(x):
    i = lax.bitcast_convert_type(x, jnp.int32)
    i = jnp.int32(0x5F3759DF) - lax.shift_right_logical(i, jnp.int32(1))
    y = lax.bitcast_convert_type(i, jnp.float32)
    for _ in range(3):
        y = y * (1.5 - 0.5 * x * y * y)
    return y


def _body(nc, spt, ids_hbm, ttb_hbm, word_hbm, pos_hbm, type_hbm, gamma_hbm,
          beta_hbm, out_hbm, idx_v, ttb_v, r0, r1, r2, r3, x_v, pos_v,
          type_v, diff_v, gamma_v, beta_v, sg0, sg1, sg2, sg3, so0, so1,
          so2, so3, sstage):
    wid = lax.axis_index("s") * nc + lax.axis_index("c")
    s0 = wid * spt
    qpb = spt // CH
    nchunks = BATCH * qpb

    id_stage = [pltpu.async_copy(ids_hbm.at[b, pl.ds(s0, spt)],
                                 idx_v.at[pl.ds(b * spt, spt)], sstage)
                for b in range(BATCH)]
    staging = [pltpu.async_copy(ttb_hbm.at[b, pl.ds(s0 * L, spt * L)],
                                ttb_v.at[b], sstage) for b in range(BATCH)]
    staging.append(pltpu.async_copy(pos_hbm.at[pl.ds(s0, spt)], pos_v,
                                    sstage))
    staging.append(pltpu.async_copy(type_hbm, type_v, sstage))
    staging.append(pltpu.async_copy(gamma_hbm, gamma_v, sstage))
    staging.append(pltpu.async_copy(beta_hbm, beta_v, sstage))
    for h in id_stage:
        h.wait()

    rows = (r0, r1, r2, r3)
    sem_g = (sg0, sg1, sg2, sg3)
    sem_o = (so0, so1, so2, so3)

    def gather(k, slot):
        b, q = k // qpb, k % qpb
        return pltpu.async_copy(
            word_hbm.at[idx_v.at[pl.ds(b * spt + q * CH, CH)]], rows[slot],
            sem_g[slot])

    gh = [None] * NBUF
    for k in range(min(NBUF - 1, BATCH * qpb)):
        gh[k] = gather(k, k)
    for h in staging:
        h.wait()

    for c in range(HC):
        o = pl.ds(c * L, L)
        diff_v[o] = type_v[1, o] - type_v[0, o]

    @plsc.parallel_loop(0, spt)
    def ploop(i):
        for g in range(0, HC, 8):
            os_ = [pl.ds(c * L, L) for c in range(g, g + 8)]
            ps = [pos_v[i, o] for o in os_]
            ts = [type_v[0, o] for o in os_]
            for o, p, t in zip(os_, ps, ts):
                pos_v[i, o] = p + t

    def compute(k, slot):
        rv = rows[slot]
        b, q = k // qpb, k % qpb
        poff = q * CH
        zero = jnp.zeros((L,), jnp.float32)
        nacc = 4

        @plsc.parallel_loop(0, CH, unroll=1)
        def tok(j):
            ttf = ttb_v[b, pl.ds((poff + j) * L, L)]

            @plsc.parallel_loop(0, HC, step=8, carry=(zero,) * (2 * nacc))
            def p1(c0, carry):
                acc = list(carry)
                b0 = c0 * L
                os_ = [pl.ds(b0 + a * L, L) for a in range(8)]
                ws = [rv[j, o] for o in os_]
                ps = [pos_v[poff + j, o] for o in os_]
                ds_ = [diff_v[o] for o in os_]
                xs = [w + p + ttf * d for w, p, d in zip(ws, ps, ds_)]
                for a, (o, x) in enumerate(zip(os_, xs)):
                    x_v[j, o] = x
                    i = a % nacc
                    acc[i] = acc[i] + x
                    acc[nacc + i] = acc[nacc + i] + x * x
                return tuple(acc)

            acc = list(p1)
            while len(acc) > 2:
                acc = ([acc[2 * i] + acc[2 * i + 1]
                        for i in range(len(acc) // 4)]
                       + [acc[len(acc) // 2 + 2 * i]
                          + acc[len(acc) // 2 + 2 * i + 1]
                          for i in range(len(acc) // 4)])
            vs, vq = acc
            mean = _hsum(vs) * (1.0 / HIDDEN)
            var = _hsum(vq) * (1.0 / HIDDEN) - mean * mean
            rstd = _rsqrt(var + EPS)
            gm = rstd * mean

            @plsc.parallel_loop(0, HC, step=8)
            def p2(c0):
                b0 = c0 * L
                os_ = [pl.ds(b0 + a * L, L) for a in range(8)]
                xs = [x_v[j, o] for o in os_]
                gs = [gamma_v[o] for o in os_]
                bs = [beta_v[o] for o in os_]
                ys = [(x * rstd - gm) * g + bb
                      for x, g, bb in zip(xs, gs, bs)]
                for o, y in zip(os_, ys):
                    rv[j, o] = y

    def writeback(k, slot):
        b, q = k // qpb, k % qpb
        return pltpu.async_copy(
            rows[slot], out_hbm.at[b, pl.ds(s0 + q * CH, CH)], sem_o[slot])

    oh = [None] * NBUF
    for k in range(nchunks):
        slot = k % NBUF
        gh[slot].wait()
        kn = k + NBUF - 1
        if kn < nchunks:
            sn = kn % NBUF
            if oh[sn] is not None:
                oh[sn].wait()
            gh[sn] = gather(kn, sn)
        compute(k, slot)
        oh[slot] = writeback(k, slot)
    for h in oh:
        if h is not None:
            h.wait()


def kernel(input_ids, token_type_ids, word_emb, pos_emb, type_emb, ln_gamma,
           ln_beta):
    ids = input_ids.astype(jnp.int32)
    ttb = jnp.broadcast_to(token_type_ids.astype(jnp.float32)[..., None],
                           (BATCH, SEQ, L)).reshape(BATCH, SEQ * L)

    try:
        info = plsc.get_sparse_core_info()
        nc, ns = info.num_cores, info.num_subcores
    except Exception:
        nc, ns = 2, 16
    nw = nc * ns
    spt = SEQ // nw

    f = pl.kernel(
        functools.partial(_body, nc, spt),
        out_type=jax.ShapeDtypeStruct((BATCH, SEQ, HIDDEN), jnp.float32),
        mesh=plsc.VectorSubcoreMesh(core_axis_name="c", subcore_axis_name="s"),
        scratch_types=[
            pltpu.VMEM((BATCH * spt,), jnp.int32),
            pltpu.VMEM((BATCH, spt * L), jnp.float32),
            pltpu.VMEM((CH, HIDDEN), jnp.float32),
            pltpu.VMEM((CH, HIDDEN), jnp.float32),
            pltpu.VMEM((CH, HIDDEN), jnp.float32),
            pltpu.VMEM((CH, HIDDEN), jnp.float32),
            pltpu.VMEM((CH, HIDDEN), jnp.float32),
            pltpu.VMEM((spt, HIDDEN), jnp.float32),
            pltpu.VMEM((2, HIDDEN), jnp.float32),
            pltpu.VMEM((HIDDEN,), jnp.float32),
            pltpu.VMEM((HIDDEN,), jnp.float32),
            pltpu.VMEM((HIDDEN,), jnp.float32),
            pltpu.SemaphoreType.DMA,
            pltpu.SemaphoreType.DMA,
            pltpu.SemaphoreType.DMA,
            pltpu.SemaphoreType.DMA,
            pltpu.SemaphoreType.DMA,
            pltpu.SemaphoreType.DMA,
            pltpu.SemaphoreType.DMA,
            pltpu.SemaphoreType.DMA,
            pltpu.SemaphoreType.DMA,
        ],
    )
    return f(ids, ttb, word_emb, pos_emb, type_emb, ln_gamma, ln_beta)
